# baseline (speedup 1.0000x reference)
import functools

import numpy as np

import jax
import jax.numpy as jnp
from jax import lax
from jax.experimental import pallas as pl
from jax.experimental.pallas import tpu as pltpu
from jax.experimental.pallas import tpu_sc as plsc

N = 10000
D = 128
E = 320000

NC = 2
NS = 16
NW = NC * NS

CH = 128
KCH = 80
EP = NW * KCH * CH
NP = 10240
RPT = NP // NS

_mesh = plsc.VectorSubcoreMesh(core_axis_name="c", subcore_axis_name="s")


DW = D


def _sc_deg_kernel(dst_chunks):
    @functools.partial(
        pl.kernel,
        out_type=jax.ShapeDtypeStruct((NC * NP, DW), jnp.float32),
        mesh=_mesh,
        scratch_types=[
            pltpu.VMEM((CH, DW), jnp.float32),
            pltpu.VMEM((CH, DW), jnp.float32),
            pltpu.VMEM((KCH, CH), jnp.int32),
            pltpu.VMEM_SHARED((NP, DW), jnp.float32),
        ],
    )
    def k(dst_hbm, out_hbm, ones_v, zero_v, dstc_v, deg_sh):
        c = lax.axis_index("c")
        s = lax.axis_index("s")
        wid = s * NC + c

        def fill(i, carry):
            for j in range(DW // 16):
                ones_v[i, pl.ds(j * 16, 16)] = jnp.ones((16,), jnp.float32)
                zero_v[i, pl.ds(j * 16, 16)] = jnp.zeros((16,), jnp.float32)
            return carry

        lax.fori_loop(0, CH, fill, None)

        for j in range(RPT // CH):
            pltpu.sync_copy(zero_v, deg_sh.at[pl.ds(s * RPT + j * CH, CH)])
        plsc.subcore_barrier()

        pltpu.sync_copy(dst_hbm.at[pl.ds(wid * KCH, KCH)], dstc_v)

        def chunk(i, carry):
            pltpu.sync_copy(ones_v, deg_sh.at[dstc_v.at[i]], add=True)
            return carry

        lax.fori_loop(0, KCH, chunk, None)
        plsc.subcore_barrier()

        pltpu.sync_copy(
            deg_sh.at[pl.ds(s * RPT, RPT)],
            out_hbm.at[pl.ds(c * NP + s * RPT, RPT)],
        )

    return k(dst_chunks)


DH = D // 2
NCHUNK = EP // CH
CPT = NCHUNK // NS
NB = 8
NSTAGE = 2
CPS = CPT // NSTAGE


def _sc_edge_kernel2(hs_sp, srcall_chunks, dst_chunks):
    @functools.partial(
        pl.kernel,
        out_type=jax.ShapeDtypeStruct((NC * NP, DH), jnp.float32),
        mesh=_mesh,
        scratch_types=[
            [pltpu.VMEM((CH, DH), jnp.float32) for _ in range(NB)],
            pltpu.VMEM((CPS, CH), jnp.int32),
            pltpu.VMEM((CPS, CH), jnp.int32),
            pltpu.VMEM_SHARED((NP, DH), jnp.float32),
            [pltpu.SemaphoreType.DMA for _ in range(NB)],
        ],
        compiler_params=pltpu.CompilerParams(use_tc_tiling_on_sc=False),
    )
    def k(hs_hbm, src_hbm, dst_hbm, out_hbm, rows, srcc_v, dstc_v, agg_sh, sems):
        c = lax.axis_index("c")
        s = lax.axis_index("s")

        def zrow(i, carry):
            for j in range(DH // 16):
                rows[0][i, pl.ds(j * 16, 16)] = jnp.zeros((16,), jnp.float32)
            return carry

        lax.fori_loop(0, CH, zrow, None)
        for j in range(RPT // CH):
            pltpu.sync_copy(rows[0], agg_sh.at[pl.ds(s * RPT + j * CH, CH)])
        plsc.subcore_barrier()

        for h in range(NSTAGE):
            base = s * CPT + h * CPS
            pltpu.sync_copy(src_hbm.at[pl.ds(c * NCHUNK + base, CPS)], srcc_v)
            pltpu.sync_copy(dst_hbm.at[pl.ds(base, CPS)], dstc_v)
            for b in range(NB):
                pltpu.async_copy(hs_hbm.at[srcc_v.at[b]], rows[b], sems[b])

            def chunk(j, carry):
                for b in range(NB):
                    i = j * NB + b
                    pltpu.make_async_copy(
                        hs_hbm.at[pl.ds(0, CH)], rows[b], sems[b]
                    ).wait()
                    pltpu.sync_copy(
                        rows[b], agg_sh.at[dstc_v.at[i]], add=True
                    )

                    @pl.when(j < CPS // NB - 1)
                    def _():
                        pltpu.async_copy(
                            hs_hbm.at[srcc_v.at[i + NB]], rows[b], sems[b]
                        )

                return carry

            lax.fori_loop(0, CPS // NB, chunk, None)
        plsc.subcore_barrier()

        pltpu.sync_copy(
            agg_sh.at[pl.ds(s * RPT, RPT)],
            out_hbm.at[pl.ds(c * NP + s * RPT, RPT)],
        )

    return k(hs_sp, srcall_chunks, dst_chunks)


def _sc_edge_kernel3(hs_bf, srcall_chunks, dst_chunks):
    @functools.partial(
        pl.kernel,
        out_type=jax.ShapeDtypeStruct((NC * NP, DH), jnp.float32),
        mesh=_mesh,
        scratch_types=[
            [pltpu.VMEM((CH, DH), jnp.bfloat16) for _ in range(NB)],
            [pltpu.VMEM((CH, DH), jnp.float32) for _ in range(2)],
            pltpu.VMEM((CPS, CH), jnp.int32),
            pltpu.VMEM((CPS, CH), jnp.int32),
            pltpu.VMEM_SHARED((NP, DH), jnp.float32),
            [pltpu.SemaphoreType.DMA for _ in range(NB)],
            [pltpu.SemaphoreType.DMA for _ in range(2)],
        ],
        compiler_params=pltpu.CompilerParams(
            use_tc_tiling_on_sc=False, needs_layout_passes=False
        ),
    )
    def k(hs_hbm, src_hbm, dst_hbm, out_hbm, rows, stages, srcc_v, dstc_v,
          agg_sh, sems, ssems):
        c = lax.axis_index("c")
        s = lax.axis_index("s")

        def zrow(i, carry):
            for j in range(DH // 16):
                stages[0][i, pl.ds(j * 16, 16)] = jnp.zeros((16,), jnp.float32)
            return carry

        lax.fori_loop(0, CH, zrow, None)
        for j in range(RPT // CH):
            pltpu.sync_copy(stages[0], agg_sh.at[pl.ds(s * RPT + j * CH, CH)])
        plsc.subcore_barrier()

        for h in range(NSTAGE):
            base = s * CPT + h * CPS
            pltpu.sync_copy(src_hbm.at[pl.ds(c * NCHUNK + base, CPS)], srcc_v)
            pltpu.sync_copy(dst_hbm.at[pl.ds(base, CPS)], dstc_v)
            for b in range(NB):
                pltpu.async_copy(hs_hbm.at[srcc_v.at[b]], rows[b], sems[b])

            def chunk(j, carry):
                for b in range(NB):
                    i = j * NB + b
                    t = b % 2
                    pltpu.make_async_copy(
                        hs_hbm.at[pl.ds(0, CH)], rows[b], sems[b]
                    ).wait()

                    def drain():
                        pltpu.make_async_copy(
                            stages[t], agg_sh.at[pl.ds(0, CH)], ssems[t]
                        ).wait()

                    if b < 2 and h == 0:
                        @pl.when(j > 0)
                        def _():
                            drain()
                    else:
                        drain()

                    def widen(r, carry2):
                        for u in range(4):
                            rr = 4 * r + u
                            for g in range(DH // 32):
                                ab = rows[b][rr, pl.ds(32 * g, 32)]
                                lo, hi = plsc.unpack(
                                    ab, format=plsc.PackFormat.INTERLEAVED
                                )
                                stages[t][rr, pl.ds(32 * g, 16)] = lo
                                stages[t][rr, pl.ds(32 * g + 16, 16)] = hi
                        return carry2

                    lax.fori_loop(0, CH // 4, widen, None)
                    pltpu.async_copy(
                        stages[t], agg_sh.at[dstc_v.at[i]], ssems[t], add=True
                    )

                    @pl.when(j < CPS // NB - 1)
                    def _():
                        pltpu.async_copy(
                            hs_hbm.at[srcc_v.at[i + NB]], rows[b], sems[b]
                        )

                return carry

            lax.fori_loop(0, CPS // NB, chunk, None)

        for t in range(2):
            pltpu.make_async_copy(
                stages[t], agg_sh.at[pl.ds(0, CH)], ssems[t]
            ).wait()
        plsc.subcore_barrier()

        pltpu.sync_copy(
            agg_sh.at[pl.ds(s * RPT, RPT)],
            out_hbm.at[pl.ds(c * NP + s * RPT, RPT)],
        )

    return k(hs_bf, srcall_chunks, dst_chunks)


_PI = np.concatenate([
    32 * b + np.concatenate([2 * np.arange(16), 2 * np.arange(16) + 1])
    for b in range(D // 32)
])


_BR = 128


def _tc_first(x, W1, W1p, degp):
    def body(x_ref, w_ref, wp_ref, degp_ref, hsb_ref, hsp_ref, dinv_ref):
        deg = degp_ref[0, :, 0:1] + degp_ref[1, :, 0:1] + 1.0
        dv = lax.rsqrt(deg)
        dvb = jnp.broadcast_to(dv, (_BR, D))
        xv = x_ref[...]
        hs = jnp.dot(xv, w_ref[...], preferred_element_type=jnp.float32) * dvb
        hsb = hs.astype(jnp.bfloat16)
        hsb_ref[0] = hsb[:, :DH]
        hsb_ref[1] = hsb[:, DH:]
        hsp = jnp.dot(xv, wp_ref[...], preferred_element_type=jnp.float32) * dvb
        hsp_ref[0] = hsp[:, :DH]
        hsp_ref[1] = hsp[:, DH:]
        dinv_ref[...] = dvb

    return pl.pallas_call(
        body,
        grid=(NP // _BR,),
        in_specs=[
            pl.BlockSpec((_BR, D), lambda i: (i, 0)),
            pl.BlockSpec((D, D), lambda i: (0, 0)),
            pl.BlockSpec((D, D), lambda i: (0, 0)),
            pl.BlockSpec((NC, _BR, DW), lambda i: (0, i, 0)),
        ],
        out_specs=[
            pl.BlockSpec((NC, _BR, DH), lambda i: (0, i, 0)),
            pl.BlockSpec((NC, _BR, DH), lambda i: (0, i, 0)),
            pl.BlockSpec((_BR, D), lambda i: (i, 0)),
        ],
        out_shape=[
            jax.ShapeDtypeStruct((NC, NP, DH), jnp.bfloat16),
            jax.ShapeDtypeStruct((NC, NP, DH), jnp.float32),
            jax.ShapeDtypeStruct((NP, D), jnp.float32),
        ],
    )(x, W1, W1p, degp)


def _tc_mid(p, hs_prev, dinv, b, Wn, Wp):
    def body(p_ref, hsp_ref, dinv_ref, b_ref, wn_ref, wp_ref, hsb_ref, out_ref):
        dv = dinv_ref[...]
        agg = jnp.concatenate([p_ref[0], p_ref[1]], axis=1)
        hsp = jnp.concatenate([hsp_ref[0], hsp_ref[1]], axis=1)
        z = dv * (agg + hsp) + b_ref[...]
        z = jnp.maximum(z, 0.0)
        hsn = jnp.dot(z, wn_ref[...], preferred_element_type=jnp.float32) * dv
        hsb = hsn.astype(jnp.bfloat16)
        hsb_ref[0] = hsb[:, :DH]
        hsb_ref[1] = hsb[:, DH:]
        hsp2 = jnp.dot(z, wp_ref[...], preferred_element_type=jnp.float32) * dv
        out_ref[0] = hsp2[:, :DH]
        out_ref[1] = hsp2[:, DH:]

    return pl.pallas_call(
        body,
        grid=(NP // _BR,),
        in_specs=[
            pl.BlockSpec((NC, _BR, DH), lambda i: (0, i, 0)),
            pl.BlockSpec((NC, _BR, DH), lambda i: (0, i, 0)),
            pl.BlockSpec((_BR, D), lambda i: (i, 0)),
            pl.BlockSpec((1, D), lambda i: (0, 0)),
            pl.BlockSpec((D, D), lambda i: (0, 0)),
            pl.BlockSpec((D, D), lambda i: (0, 0)),
        ],
        out_specs=[
            pl.BlockSpec((NC, _BR, DH), lambda i: (0, i, 0)),
            pl.BlockSpec((NC, _BR, DH), lambda i: (0, i, 0)),
        ],
        out_shape=[
            jax.ShapeDtypeStruct((NC, NP, DH), jnp.bfloat16),
            jax.ShapeDtypeStruct((NC, NP, DH), jnp.float32),
        ],
    )(p, hs_prev, dinv, b, Wn, Wp)


def _tc_final(p, hs3, dinv, b, pmat):
    def body(p_ref, hs_ref, dinv_ref, b_ref, pm_ref, out_ref):
        agg = jnp.concatenate([p_ref[0], p_ref[1]], axis=1)
        hsv = jnp.concatenate([hs_ref[0], hs_ref[1]], axis=1)
        z = dinv_ref[...] * (agg + hsv) + b_ref[...]
        m = jnp.max(z, axis=1, keepdims=True)
        ez = jnp.exp(z - m)
        sz = jnp.sum(ez, axis=1, keepdims=True)
        lsm = z - m - jnp.log(sz)
        out_ref[...] = jnp.dot(
            lsm, pm_ref[...], preferred_element_type=jnp.float32
        )

    return pl.pallas_call(
        body,
        grid=(NP // _BR,),
        in_specs=[
            pl.BlockSpec((NC, _BR, DH), lambda i: (0, i, 0)),
            pl.BlockSpec((NC, _BR, DH), lambda i: (0, i, 0)),
            pl.BlockSpec((_BR, D), lambda i: (i, 0)),
            pl.BlockSpec((1, D), lambda i: (0, 0)),
            pl.BlockSpec((D, D), lambda i: (0, 0)),
        ],
        out_specs=pl.BlockSpec((_BR, D), lambda i: (i, 0)),
        out_shape=jax.ShapeDtypeStruct((NP, D), jnp.float32),
    )(p, hs3, dinv, b, pmat)


@jax.jit
def kernel(x, edge_index, W1, b1, W2, b2, W3, b3):
    src = edge_index[0]
    dst = edge_index[1]
    pad_e = EP - E
    src_f = jnp.concatenate([src, jnp.full((pad_e,), N, jnp.int32)])
    dst_p = jnp.concatenate([dst, jnp.full((pad_e,), N, jnp.int32)]).reshape(
        NCHUNK, CH
    )
    srcall = jnp.concatenate([src_f, src_f + NP]).reshape(NC * NCHUNK, CH)
    x_p = jnp.zeros((NP, D), jnp.float32).at[:N].set(x)
    b1r = b1.reshape(1, D)
    b2r = b2.reshape(1, D)
    b3r = b3.reshape(1, D)

    W1p = W1[:, _PI]
    W2n = W2[_PI, :]
    W2p = W2n[:, _PI]
    W3n = W3[_PI, :]
    W3p = W3n[:, _PI]
    b1p = b1r[:, _PI]
    b2p = b2r[:, _PI]
    b3p = b3r[:, _PI]
    pm = np.zeros((D, D), np.float32)
    pm[np.arange(D), _PI] = 1.0
    pmat = jnp.asarray(pm)

    degp = _sc_deg_kernel(dst_p).reshape(NC, NP, DW)
    hs1b, hs1p, dinv = _tc_first(x_p, W1, W1p, degp)

    p1 = _sc_edge_kernel3(hs1b.reshape(NC * NP, DH), srcall, dst_p)
    hs2b, hs2p = _tc_mid(p1.reshape(NC, NP, DH), hs1p, dinv, b1p, W2n, W2p)

    p2 = _sc_edge_kernel3(hs2b.reshape(NC * NP, DH), srcall, dst_p)
    hs3b, hs3p = _tc_mid(p2.reshape(NC, NP, DH), hs2p, dinv, b2p, W3n, W3p)

    p3 = _sc_edge_kernel3(hs3b.reshape(NC * NP, DH), srcall, dst_p)
    out = _tc_final(p3.reshape(NC, NP, DH), hs3p, dinv, b3p, pmat)
    return out[:N]


# speedup vs baseline: 1.0144x; 1.0144x over previous
import functools

import numpy as np

import jax
import jax.numpy as jnp
from jax import lax
from jax.experimental import pallas as pl
from jax.experimental.pallas import tpu as pltpu
from jax.experimental.pallas import tpu_sc as plsc

N = 10000
D = 128
E = 320000

NC = 2
NS = 16
NW = NC * NS

CH = 128
KCH = 80
EP = NW * KCH * CH
NP = 10240
RPT = NP // NS

_mesh = plsc.VectorSubcoreMesh(core_axis_name="c", subcore_axis_name="s")


DW = D


def _sc_deg_kernel(dst_chunks):
    @functools.partial(
        pl.kernel,
        out_type=jax.ShapeDtypeStruct((NC * NP, DW), jnp.float32),
        mesh=_mesh,
        scratch_types=[
            pltpu.VMEM((CH, DW), jnp.float32),
            pltpu.VMEM((CH, DW), jnp.float32),
            pltpu.VMEM((KCH, CH), jnp.int32),
            pltpu.VMEM_SHARED((NP, DW), jnp.float32),
        ],
    )
    def k(dst_hbm, out_hbm, ones_v, zero_v, dstc_v, deg_sh):
        c = lax.axis_index("c")
        s = lax.axis_index("s")
        wid = s * NC + c

        def fill(i, carry):
            for j in range(DW // 16):
                ones_v[i, pl.ds(j * 16, 16)] = jnp.ones((16,), jnp.float32)
                zero_v[i, pl.ds(j * 16, 16)] = jnp.zeros((16,), jnp.float32)
            return carry

        lax.fori_loop(0, CH, fill, None)

        for j in range(RPT // CH):
            pltpu.sync_copy(zero_v, deg_sh.at[pl.ds(s * RPT + j * CH, CH)])
        plsc.subcore_barrier()

        pltpu.sync_copy(dst_hbm.at[pl.ds(wid * KCH, KCH)], dstc_v)

        def chunk(i, carry):
            pltpu.sync_copy(ones_v, deg_sh.at[dstc_v.at[i]], add=True)
            return carry

        lax.fori_loop(0, KCH, chunk, None)
        plsc.subcore_barrier()

        pltpu.sync_copy(
            deg_sh.at[pl.ds(s * RPT, RPT)],
            out_hbm.at[pl.ds(c * NP + s * RPT, RPT)],
        )

    return k(dst_chunks)


DH = D // 2
NCHUNK = EP // CH
CPT = NCHUNK // NS
NB = 8
NSTAGE = 2
CPS = CPT // NSTAGE


def _sc_edge_kernel3(hs_bf, srcall_chunks, dst_chunks):
    @functools.partial(
        pl.kernel,
        out_type=jax.ShapeDtypeStruct((NC * NP, DH), jnp.float32),
        mesh=_mesh,
        scratch_types=[
            [pltpu.VMEM((CH, DH), jnp.bfloat16) for _ in range(NB)],
            [pltpu.VMEM((CH, DH), jnp.float32) for _ in range(2)],
            pltpu.VMEM((CPS, CH), jnp.int32),
            pltpu.VMEM((CPS, CH), jnp.int32),
            pltpu.VMEM_SHARED((NP, DH), jnp.float32),
            [pltpu.SemaphoreType.DMA for _ in range(NB)],
            [pltpu.SemaphoreType.DMA for _ in range(2)],
        ],
        compiler_params=pltpu.CompilerParams(
            use_tc_tiling_on_sc=False, needs_layout_passes=False
        ),
    )
    def k(hs_hbm, src_hbm, dst_hbm, out_hbm, rows, stages, srcc_v, dstc_v,
          agg_sh, sems, ssems):
        c = lax.axis_index("c")
        s = lax.axis_index("s")

        def zrow(i, carry):
            for j in range(DH // 16):
                stages[0][i, pl.ds(j * 16, 16)] = jnp.zeros((16,), jnp.float32)
            return carry

        lax.fori_loop(0, CH, zrow, None)
        for j in range(RPT // CH):
            pltpu.sync_copy(stages[0], agg_sh.at[pl.ds(s * RPT + j * CH, CH)])
        plsc.subcore_barrier()

        for h in range(NSTAGE):
            base = s * CPT + h * CPS
            pltpu.sync_copy(src_hbm.at[pl.ds(c * NCHUNK + base, CPS)], srcc_v)
            pltpu.sync_copy(dst_hbm.at[pl.ds(base, CPS)], dstc_v)
            for b in range(NB):
                pltpu.async_copy(hs_hbm.at[srcc_v.at[b]], rows[b], sems[b])

            def chunk(j, carry):
                for b in range(NB):
                    i = j * NB + b
                    t = b % 2
                    pltpu.make_async_copy(
                        hs_hbm.at[pl.ds(0, CH)], rows[b], sems[b]
                    ).wait()

                    def drain():
                        pltpu.make_async_copy(
                            stages[t], agg_sh.at[pl.ds(0, CH)], ssems[t]
                        ).wait()

                    if b < 2 and h == 0:
                        @pl.when(j > 0)
                        def _():
                            drain()
                    else:
                        drain()

                    def widen(r, carry2):
                        for rr in (2 * r, 2 * r + 1):
                            for g in range(DH // 32):
                                ab = rows[b][rr, pl.ds(32 * g, 32)]
                                lo, hi = plsc.unpack(
                                    ab, format=plsc.PackFormat.INTERLEAVED
                                )
                                stages[t][rr, pl.ds(32 * g, 16)] = lo
                                stages[t][rr, pl.ds(32 * g + 16, 16)] = hi
                        return carry2

                    lax.fori_loop(0, CH // 2, widen, None)
                    pltpu.async_copy(
                        stages[t], agg_sh.at[dstc_v.at[i]], ssems[t], add=True
                    )

                    @pl.when(j < CPS // NB - 1)
                    def _():
                        pltpu.async_copy(
                            hs_hbm.at[srcc_v.at[i + NB]], rows[b], sems[b]
                        )

                return carry

            lax.fori_loop(0, CPS // NB, chunk, None)

        for t in range(2):
            pltpu.make_async_copy(
                stages[t], agg_sh.at[pl.ds(0, CH)], ssems[t]
            ).wait()
        plsc.subcore_barrier()

        pltpu.sync_copy(
            agg_sh.at[pl.ds(s * RPT, RPT)],
            out_hbm.at[pl.ds(c * NP + s * RPT, RPT)],
        )

    return k(hs_bf, srcall_chunks, dst_chunks)


_PI = np.concatenate([
    32 * b + np.concatenate([2 * np.arange(16), 2 * np.arange(16) + 1])
    for b in range(D // 32)
])


_BR = 128


def _tc_first(x, W1, W1p, degp):
    def body(x_ref, w_ref, wp_ref, degp_ref, hsb_ref, hsp_ref, dinv_ref):
        deg = degp_ref[0, :, 0:1] + degp_ref[1, :, 0:1] + 1.0
        dv = lax.rsqrt(deg)
        dvb = jnp.broadcast_to(dv, (_BR, D))
        xv = x_ref[...]
        hs = jnp.dot(xv, w_ref[...], preferred_element_type=jnp.float32) * dvb
        hsb = hs.astype(jnp.bfloat16)
        hsb_ref[0] = hsb[:, :DH]
        hsb_ref[1] = hsb[:, DH:]
        hsp = jnp.dot(xv, wp_ref[...], preferred_element_type=jnp.float32) * dvb
        hsp_ref[0] = hsp[:, :DH]
        hsp_ref[1] = hsp[:, DH:]
        dinv_ref[...] = dvb

    return pl.pallas_call(
        body,
        grid=(NP // _BR,),
        in_specs=[
            pl.BlockSpec((_BR, D), lambda i: (i, 0)),
            pl.BlockSpec((D, D), lambda i: (0, 0)),
            pl.BlockSpec((D, D), lambda i: (0, 0)),
            pl.BlockSpec((NC, _BR, DW), lambda i: (0, i, 0)),
        ],
        out_specs=[
            pl.BlockSpec((NC, _BR, DH), lambda i: (0, i, 0)),
            pl.BlockSpec((NC, _BR, DH), lambda i: (0, i, 0)),
            pl.BlockSpec((_BR, D), lambda i: (i, 0)),
        ],
        out_shape=[
            jax.ShapeDtypeStruct((NC, NP, DH), jnp.bfloat16),
            jax.ShapeDtypeStruct((NC, NP, DH), jnp.float32),
            jax.ShapeDtypeStruct((NP, D), jnp.float32),
        ],
    )(x, W1, W1p, degp)


def _tc_mid(p, hs_prev, dinv, b, Wn, Wp):
    def body(p_ref, hsp_ref, dinv_ref, b_ref, wn_ref, wp_ref, hsb_ref, out_ref):
        dv = dinv_ref[...]
        agg = jnp.concatenate([p_ref[0], p_ref[1]], axis=1)
        hsp = jnp.concatenate([hsp_ref[0], hsp_ref[1]], axis=1)
        z = dv * (agg + hsp) + b_ref[...]
        z = jnp.maximum(z, 0.0)
        hsn = jnp.dot(z, wn_ref[...], preferred_element_type=jnp.float32) * dv
        hsb = hsn.astype(jnp.bfloat16)
        hsb_ref[0] = hsb[:, :DH]
        hsb_ref[1] = hsb[:, DH:]
        hsp2 = jnp.dot(z, wp_ref[...], preferred_element_type=jnp.float32) * dv
        out_ref[0] = hsp2[:, :DH]
        out_ref[1] = hsp2[:, DH:]

    return pl.pallas_call(
        body,
        grid=(NP // _BR,),
        in_specs=[
            pl.BlockSpec((NC, _BR, DH), lambda i: (0, i, 0)),
            pl.BlockSpec((NC, _BR, DH), lambda i: (0, i, 0)),
            pl.BlockSpec((_BR, D), lambda i: (i, 0)),
            pl.BlockSpec((1, D), lambda i: (0, 0)),
            pl.BlockSpec((D, D), lambda i: (0, 0)),
            pl.BlockSpec((D, D), lambda i: (0, 0)),
        ],
        out_specs=[
            pl.BlockSpec((NC, _BR, DH), lambda i: (0, i, 0)),
            pl.BlockSpec((NC, _BR, DH), lambda i: (0, i, 0)),
        ],
        out_shape=[
            jax.ShapeDtypeStruct((NC, NP, DH), jnp.bfloat16),
            jax.ShapeDtypeStruct((NC, NP, DH), jnp.float32),
        ],
    )(p, hs_prev, dinv, b, Wn, Wp)


def _tc_final(p, hs3, dinv, b, pmat):
    def body(p_ref, hs_ref, dinv_ref, b_ref, pm_ref, out_ref):
        agg = jnp.concatenate([p_ref[0], p_ref[1]], axis=1)
        hsv = jnp.concatenate([hs_ref[0], hs_ref[1]], axis=1)
        z = dinv_ref[...] * (agg + hsv) + b_ref[...]
        m = jnp.max(z, axis=1, keepdims=True)
        ez = jnp.exp(z - m)
        sz = jnp.sum(ez, axis=1, keepdims=True)
        lsm = z - m - jnp.log(sz)
        out_ref[...] = jnp.dot(
            lsm, pm_ref[...], preferred_element_type=jnp.float32
        )

    return pl.pallas_call(
        body,
        grid=(NP // _BR,),
        in_specs=[
            pl.BlockSpec((NC, _BR, DH), lambda i: (0, i, 0)),
            pl.BlockSpec((NC, _BR, DH), lambda i: (0, i, 0)),
            pl.BlockSpec((_BR, D), lambda i: (i, 0)),
            pl.BlockSpec((1, D), lambda i: (0, 0)),
            pl.BlockSpec((D, D), lambda i: (0, 0)),
        ],
        out_specs=pl.BlockSpec((_BR, D), lambda i: (i, 0)),
        out_shape=jax.ShapeDtypeStruct((NP, D), jnp.float32),
    )(p, hs3, dinv, b, pmat)


@jax.jit
def kernel(x, edge_index, W1, b1, W2, b2, W3, b3):
    src = edge_index[0]
    dst = edge_index[1]
    pad_e = EP - E
    src_f = jnp.concatenate([src, jnp.full((pad_e,), N, jnp.int32)])
    dst_p = jnp.concatenate([dst, jnp.full((pad_e,), N, jnp.int32)]).reshape(
        NCHUNK, CH
    )
    srcall = jnp.concatenate([src_f, src_f + NP]).reshape(NC * NCHUNK, CH)
    x_p = jnp.zeros((NP, D), jnp.float32).at[:N].set(x)
    b1r = b1.reshape(1, D)
    b2r = b2.reshape(1, D)
    b3r = b3.reshape(1, D)

    W1p = W1[:, _PI]
    W2n = W2[_PI, :]
    W2p = W2n[:, _PI]
    W3n = W3[_PI, :]
    W3p = W3n[:, _PI]
    b1p = b1r[:, _PI]
    b2p = b2r[:, _PI]
    b3p = b3r[:, _PI]
    pm = np.zeros((D, D), np.float32)
    pm[np.arange(D), _PI] = 1.0
    pmat = jnp.asarray(pm)

    degp = _sc_deg_kernel(dst_p).reshape(NC, NP, DW)
    hs1b, hs1p, dinv = _tc_first(x_p, W1, W1p, degp)

    p1 = _sc_edge_kernel3(hs1b.reshape(NC * NP, DH), srcall, dst_p)
    hs2b, hs2p = _tc_mid(p1.reshape(NC, NP, DH), hs1p, dinv, b1p, W2n, W2p)

    p2 = _sc_edge_kernel3(hs2b.reshape(NC * NP, DH), srcall, dst_p)
    hs3b, hs3p = _tc_mid(p2.reshape(NC, NP, DH), hs2p, dinv, b2p, W3n, W3p)

    p3 = _sc_edge_kernel3(hs3b.reshape(NC * NP, DH), srcall, dst_p)
    out = _tc_final(p3.reshape(NC, NP, DH), hs3p, dinv, b3p, pmat)
    return out[:N]
